# baseline (speedup 1.0000x reference)
import functools

import jax
import jax.numpy as jnp
from jax import lax
from jax.experimental import pallas as pl
from jax.experimental.pallas import tpu as pltpu
from jax.experimental.pallas import tpu_sc as plsc

D = 768
E = 8
H = 4 * D
N = 2048
BN = 256
P = 6144
NB = P // BN
HBS = 1536
HB = H // HBS

NW = 32
NC = 2
TN = N // NW


def _sigmoid(v):
    return 1.0 / (1.0 + jnp.exp(-v))


def _dotT(a, b):
    return jax.lax.dot_general(a, b, (((1,), (1,)), ((), ())),
                               preferred_element_type=jnp.float32)


def _router_body(x_r, gw_r, i0_r, i1_r, w0_r, w1_r, hist_r, aux_r, blke_r):
    logits = _dotT(x_r[...], gw_r[...])
    col = jax.lax.broadcasted_iota(jnp.int32, (N, E), 1)
    m1 = jnp.max(logits, axis=1, keepdims=True)
    i1 = jnp.min(jnp.where(logits == m1, col, E), axis=1, keepdims=True)
    masked = jnp.where(col == i1, -jnp.inf, logits)
    m2 = jnp.max(masked, axis=1, keepdims=True)
    i2 = jnp.min(jnp.where(masked == m2, col, E), axis=1, keepdims=True)
    p1 = _sigmoid(m1 - m2)
    i0_r[...] = i1
    i1_r[...] = i2
    w0_r[...] = p1 + jnp.zeros((N, 128), jnp.float32)
    w1_r[...] = (1.0 - p1) + jnp.zeros((N, 128), jnp.float32)

    col16 = jax.lax.broadcasted_iota(jnp.int32, (N, 16), 1)
    oh = (jnp.where(col16 == i1, 1.0, 0.0)
          + jnp.where(col16 == i2, 1.0, 0.0))
    rowi = jax.lax.broadcasted_iota(jnp.int32, (NW, N), 0)
    coln = jax.lax.broadcasted_iota(jnp.int32, (NW, N), 1)
    S = jnp.where((coln // TN) == rowi, 1.0, 0.0)
    hist = jax.lax.dot_general(S, oh, (((1,), (0,)), ((), ())),
                               preferred_element_type=jnp.float32)
    hist_r[...] = hist.astype(jnp.int32)

    cnt = jnp.sum(oh, axis=0, keepdims=True)
    nblk = jnp.floor((cnt + (BN - 1)) * (1.0 / BN))
    tri = jnp.where(
        jax.lax.broadcasted_iota(jnp.int32, (16, 16), 0)
        <= jax.lax.broadcasted_iota(jnp.int32, (16, 16), 1), 1.0, 0.0)
    incl = jax.lax.dot_general(nblk, tri, (((1,), (0,)), ((), ())),
                               preferred_element_type=jnp.float32)
    base_row = (incl - nblk) * BN
    aux = jnp.concatenate([cnt, nblk, base_row, incl,
                           jnp.zeros((4, 16), jnp.float32)], axis=0)
    aux_r[...] = aux.astype(jnp.int32)

    bcols = jax.lax.broadcasted_iota(jnp.int32, (8, NB), 1).astype(jnp.float32)
    acc = jnp.zeros((8, NB), jnp.float32)
    for e in range(E):
        acc = acc + jnp.where(bcols >= incl[0, e], 1.0, 0.0)
    blke_r[...] = jnp.minimum(acc, E - 1).astype(jnp.int32)


def _router(xf, gate_w):
    return pl.pallas_call(
        _router_body,
        out_shape=(
            jax.ShapeDtypeStruct((N, 1), jnp.int32),
            jax.ShapeDtypeStruct((N, 1), jnp.int32),
            jax.ShapeDtypeStruct((N, 128), jnp.float32),
            jax.ShapeDtypeStruct((N, 128), jnp.float32),
            jax.ShapeDtypeStruct((NW, 16), jnp.int32),
            jax.ShapeDtypeStruct((8, 16), jnp.int32),
            jax.ShapeDtypeStruct((8, NB), jnp.int32),
        ),
    )(xf, gate_w)


_sc_mesh = plsc.VectorSubcoreMesh(core_axis_name="c", subcore_axis_name="s")


@functools.partial(
    pl.kernel,
    out_type=(
        jax.ShapeDtypeStruct((P, D), jnp.float32),
        jax.ShapeDtypeStruct((N,), jnp.int32),
        jax.ShapeDtypeStruct((N,), jnp.int32),
        jax.ShapeDtypeStruct((P, 128), jnp.float32),
    ),
    mesh=_sc_mesh,
    scratch_types=[
        pltpu.VMEM((TN,), jnp.int32),
        pltpu.VMEM((TN,), jnp.int32),
        pltpu.VMEM((NW, 16), jnp.int32),
        pltpu.VMEM((8, 16), jnp.int32),
        pltpu.VMEM((16,), jnp.int32),
        pltpu.VMEM((TN,), jnp.int32),
        pltpu.VMEM((TN,), jnp.int32),
        pltpu.VMEM((TN,), jnp.int32),
        pltpu.VMEM((TN, D), jnp.float32),
        pltpu.VMEM((TN, 128), jnp.float32),
        pltpu.VMEM((TN, 128), jnp.float32),
        pltpu.SemaphoreType.DMA,
        pltpu.SemaphoreType.DMA,
    ],
    compiler_params=pltpu.CompilerParams(needs_layout_passes=False),
)
def _dispatch(i0_h, i1_h, x_h, hist_h, aux_h, w0_h, w1_h,
              xd_h, s0_h, s1_h, wd_h,
              e0_v, e1_v, hist_v, aux_v, start_v, sl0_v, sl1_v, tok_v,
              rows_v, wv_v, wv2_v, sem, semw):
    wid = lax.axis_index("s") * NC + lax.axis_index("c")
    base = wid * TN
    pltpu.sync_copy(i0_h.at[pl.ds(base, TN)], e0_v)
    pltpu.sync_copy(i1_h.at[pl.ds(base, TN)], e1_v)
    pltpu.sync_copy(hist_h, hist_v)
    pltpu.sync_copy(aux_h, aux_v)

    lane = lax.iota(jnp.int32, 16)
    widv = jnp.full((16,), wid, jnp.int32)
    zero = jnp.zeros((16,), jnp.int32)
    one = jnp.full((16,), 1, jnp.int32)

    pre = zero
    for w2 in range(NW):
        m = jnp.full((16,), w2, jnp.int32) < widv
        pre = pre + jnp.where(m, hist_v[w2], zero)
    start_v[...] = aux_v[2] + pre

    for ev, slv in ((e0_v, sl0_v), (e1_v, sl1_v)):
        for r in range(TN // 16):
            v = ev[pl.ds(r * 16, 16)]
            sg = plsc.load_gather(start_v, [v])
            start = start_v[...]
            rank = zero
            for e in range(E):
                m = v == jnp.int32(e)
                c = plsc.cumsum(jnp.where(m, one, zero))
                rank = jnp.where(m, c - 1, rank)
                pc = plsc.all_reduce_population_count(m)
                start = jnp.where(lane == jnp.int32(e), start + pc, start)
            start_v[...] = start
            slv[pl.ds(r * 16, 16)] = sg + rank

    for r in range(TN // 16):
        tok_v[pl.ds(r * 16, 16)] = jnp.full((16,), base + r * 16, jnp.int32) + lane
    cpx = pltpu.async_copy(x_h.at[tok_v], rows_v, sem)
    cpw = pltpu.async_copy(w0_h.at[tok_v], wv_v, semw)
    cpx.wait()
    cs0 = pltpu.async_copy(rows_v, xd_h.at[sl0_v], sem)
    cs1 = pltpu.async_copy(rows_v, xd_h.at[sl1_v], sem)
    cpw.wait()
    cw0 = pltpu.async_copy(wv_v, wd_h.at[sl0_v], semw)
    cw0.wait()
    cpw2 = pltpu.async_copy(w1_h.at[tok_v], wv2_v, semw)
    cpw2.wait()
    cw1 = pltpu.async_copy(wv2_v, wd_h.at[sl1_v], semw)
    cs0.wait()
    cs1.wait()
    cw1.wait()

    pltpu.sync_copy(sl0_v, s0_h.at[pl.ds(base, TN)])
    pltpu.sync_copy(sl1_v, s1_h.at[pl.ds(base, TN)])


def _ffn_body(be_ref, xd_r, wi_r, wg_r, bi_r, bg_r, wo_r, bo_r, wd_r, out_r):
    hb = pl.program_id(0)
    nb = pl.program_id(1)
    xv = xd_r[...].astype(jnp.bfloat16)
    g = (_dotT(xv, wg_r[0].astype(jnp.bfloat16)) + bg_r[0]).astype(jnp.bfloat16)
    p = (_dotT(xv, wi_r[0].astype(jnp.bfloat16)) + bi_r[0]).astype(jnp.bfloat16)
    h = (g * _sigmoid(g)) * p
    y = _dotT(h, wo_r[0].astype(jnp.bfloat16))
    sl = pl.ds(nb * BN, BN)

    wv = wd_r[:, :1]

    @pl.when(hb == 0)
    def _init():
        out_r[sl, :] = wv * (y + bo_r[0])

    @pl.when(hb > 0)
    def _acc():
        out_r[sl, :] += wv * y


def _grouped_ffn(blk_e, xd, W_in, b_in, W_gate, b_gate, W_out, b_out, wd):
    spec = pltpu.PrefetchScalarGridSpec(
        num_scalar_prefetch=1,
        grid=(HB, NB),
        in_specs=[
            pl.BlockSpec((BN, D), lambda hb, nb, be: (nb, 0)),
            pl.BlockSpec((1, HBS, D), lambda hb, nb, be: (be[nb], hb, 0)),
            pl.BlockSpec((1, HBS, D), lambda hb, nb, be: (be[nb], hb, 0)),
            pl.BlockSpec((1, 1, HBS), lambda hb, nb, be: (be[nb], 0, hb)),
            pl.BlockSpec((1, 1, HBS), lambda hb, nb, be: (be[nb], 0, hb)),
            pl.BlockSpec((1, D, HBS), lambda hb, nb, be: (be[nb], 0, hb)),
            pl.BlockSpec((1, 1, D), lambda hb, nb, be: (be[nb], 0, 0)),
            pl.BlockSpec((BN, 128), lambda hb, nb, be: (nb, 0)),
        ],
        out_specs=pl.BlockSpec((P, D), lambda hb, nb, be: (0, 0)),
    )
    return pl.pallas_call(
        _ffn_body, grid_spec=spec,
        out_shape=jax.ShapeDtypeStruct((P, D), jnp.float32),
    )(blk_e, xd, W_in, W_gate, b_in.reshape(E, 1, H), b_gate.reshape(E, 1, H),
      W_out, b_out.reshape(E, 1, D), wd)


@functools.partial(
    pl.kernel,
    out_type=jax.ShapeDtypeStruct((N, D), jnp.float32),
    mesh=_sc_mesh,
    scratch_types=[
        pltpu.VMEM((TN,), jnp.int32),
        pltpu.VMEM((TN,), jnp.int32),
        pltpu.VMEM((TN, D), jnp.float32),
        pltpu.VMEM((TN, D), jnp.float32),
        pltpu.SemaphoreType.DMA,
        pltpu.SemaphoreType.DMA,
    ],
    compiler_params=pltpu.CompilerParams(needs_layout_passes=False),
)
def _combine(yd_h, s0_h, s1_h, out_h, s0_v, s1_v, r0_v, r1_v, sem0, sem1):
    wid = lax.axis_index("s") * NC + lax.axis_index("c")
    base = wid * TN
    pltpu.sync_copy(s0_h.at[pl.ds(base, TN)], s0_v)
    pltpu.sync_copy(s1_h.at[pl.ds(base, TN)], s1_v)
    cp0 = pltpu.async_copy(yd_h.at[s0_v], r0_v, sem0)
    cp1 = pltpu.async_copy(yd_h.at[s1_v], r1_v, sem1)
    cp0.wait()
    cp1.wait()

    def tok_body(t, carry):
        def ch_body(c, inner):
            s = pl.ds(c * 16, 16)
            plsc.addupdate(r0_v.at[t, s], r1_v[t, s])
            return inner

        return jax.lax.fori_loop(0, D // 16, ch_body, carry)

    jax.lax.fori_loop(0, TN, tok_body, 0)
    pltpu.sync_copy(r0_v, out_h.at[pl.ds(base, TN)])


def kernel(x, gate_w, W_in, b_in, W_gate, b_gate, W_out, b_out):
    B, T, C = x.shape
    xf = x.reshape(B * T, C)
    i0, i1, w0, w1, hist, aux, blkeT = _router(xf, gate_w)
    xd, s0, s1, wd = _dispatch(i0.reshape(N), i1.reshape(N), xf, hist, aux,
                               w0, w1)
    yd = _grouped_ffn(blkeT[0], xd, W_in, b_in, W_gate, b_gate, W_out, b_out,
                      wd)
    out = _combine(yd, s0, s1)
    return out.reshape(B, T, C)


# speedup vs baseline: 1.0373x; 1.0373x over previous
import functools

import jax
import jax.numpy as jnp
from jax import lax
from jax.experimental import pallas as pl
from jax.experimental.pallas import tpu as pltpu
from jax.experimental.pallas import tpu_sc as plsc

D = 768
E = 8
H = 4 * D
N = 2048
BN = 256
P = 6144
NB = P // BN
HBS = 1536
HB = H // HBS

NW = 32
NC = 2
TN = N // NW


def _sigmoid(v):
    return 1.0 / (1.0 + jnp.exp(-v))


def _dotT(a, b):
    return jax.lax.dot_general(a, b, (((1,), (1,)), ((), ())),
                               preferred_element_type=jnp.float32)


def _router_body(x_r, gw_r, i0_r, i1_r, w0_r, w1_r, hist_r, aux_r, blke_r):
    logits = _dotT(x_r[...], gw_r[...])
    col = jax.lax.broadcasted_iota(jnp.int32, (N, E), 1)
    m1 = jnp.max(logits, axis=1, keepdims=True)
    i1 = jnp.min(jnp.where(logits == m1, col, E), axis=1, keepdims=True)
    masked = jnp.where(col == i1, -jnp.inf, logits)
    m2 = jnp.max(masked, axis=1, keepdims=True)
    i2 = jnp.min(jnp.where(masked == m2, col, E), axis=1, keepdims=True)
    p1 = _sigmoid(m1 - m2)
    i0_r[...] = i1
    i1_r[...] = i2
    w0_r[...] = p1 + jnp.zeros((N, 128), jnp.float32)
    w1_r[...] = (1.0 - p1) + jnp.zeros((N, 128), jnp.float32)

    col16 = jax.lax.broadcasted_iota(jnp.int32, (N, 16), 1)
    oh = (jnp.where(col16 == i1, 1.0, 0.0)
          + jnp.where(col16 == i2, 1.0, 0.0))
    rowi = jax.lax.broadcasted_iota(jnp.int32, (NW, N), 0)
    coln = jax.lax.broadcasted_iota(jnp.int32, (NW, N), 1)
    S = jnp.where((coln // TN) == rowi, 1.0, 0.0)
    hist = jax.lax.dot_general(S, oh, (((1,), (0,)), ((), ())),
                               preferred_element_type=jnp.float32)
    hist_r[...] = hist.astype(jnp.int32)

    cnt = jnp.sum(oh, axis=0, keepdims=True)
    nblk = jnp.floor((cnt + (BN - 1)) * (1.0 / BN))
    tri = jnp.where(
        jax.lax.broadcasted_iota(jnp.int32, (16, 16), 0)
        <= jax.lax.broadcasted_iota(jnp.int32, (16, 16), 1), 1.0, 0.0)
    incl = jax.lax.dot_general(nblk, tri, (((1,), (0,)), ((), ())),
                               preferred_element_type=jnp.float32)
    base_row = (incl - nblk) * BN
    aux = jnp.concatenate([cnt, nblk, base_row, incl,
                           jnp.zeros((4, 16), jnp.float32)], axis=0)
    aux_r[...] = aux.astype(jnp.int32)

    bcols = jax.lax.broadcasted_iota(jnp.int32, (8, NB), 1).astype(jnp.float32)
    acc = jnp.zeros((8, NB), jnp.float32)
    for e in range(E):
        acc = acc + jnp.where(bcols >= incl[0, e], 1.0, 0.0)
    blke_r[...] = jnp.minimum(acc, E - 1).astype(jnp.int32)


def _router(xf, gate_w):
    return pl.pallas_call(
        _router_body,
        out_shape=(
            jax.ShapeDtypeStruct((N, 1), jnp.int32),
            jax.ShapeDtypeStruct((N, 1), jnp.int32),
            jax.ShapeDtypeStruct((N, 128), jnp.float32),
            jax.ShapeDtypeStruct((N, 128), jnp.float32),
            jax.ShapeDtypeStruct((NW, 16), jnp.int32),
            jax.ShapeDtypeStruct((8, 16), jnp.int32),
            jax.ShapeDtypeStruct((8, NB), jnp.int32),
        ),
    )(xf, gate_w)


_sc_mesh = plsc.VectorSubcoreMesh(core_axis_name="c", subcore_axis_name="s")


@functools.partial(
    pl.kernel,
    out_type=(
        jax.ShapeDtypeStruct((P, D), jnp.float32),
        jax.ShapeDtypeStruct((N,), jnp.int32),
        jax.ShapeDtypeStruct((N,), jnp.int32),
        jax.ShapeDtypeStruct((P, 128), jnp.float32),
    ),
    mesh=_sc_mesh,
    scratch_types=[
        pltpu.VMEM((TN,), jnp.int32),
        pltpu.VMEM((TN,), jnp.int32),
        pltpu.VMEM((NW, 16), jnp.int32),
        pltpu.VMEM((8, 16), jnp.int32),
        pltpu.VMEM((16,), jnp.int32),
        pltpu.VMEM((TN,), jnp.int32),
        pltpu.VMEM((TN,), jnp.int32),
        pltpu.VMEM((TN,), jnp.int32),
        pltpu.VMEM((TN, D), jnp.float32),
        pltpu.VMEM((TN, 128), jnp.float32),
        pltpu.VMEM((TN, 128), jnp.float32),
        pltpu.SemaphoreType.DMA,
        pltpu.SemaphoreType.DMA,
    ],
    compiler_params=pltpu.CompilerParams(needs_layout_passes=False),
)
def _dispatch(i0_h, i1_h, x_h, hist_h, aux_h, w0_h, w1_h,
              xd_h, s0_h, s1_h, wd_h,
              e0_v, e1_v, hist_v, aux_v, start_v, sl0_v, sl1_v, tok_v,
              rows_v, wv_v, wv2_v, sem, semw):
    wid = lax.axis_index("s") * NC + lax.axis_index("c")
    base = wid * TN
    pltpu.sync_copy(i0_h.at[pl.ds(base, TN)], e0_v)
    pltpu.sync_copy(i1_h.at[pl.ds(base, TN)], e1_v)
    pltpu.sync_copy(hist_h, hist_v)
    pltpu.sync_copy(aux_h, aux_v)

    lane = lax.iota(jnp.int32, 16)
    widv = jnp.full((16,), wid, jnp.int32)
    zero = jnp.zeros((16,), jnp.int32)
    one = jnp.full((16,), 1, jnp.int32)

    pre = zero
    for w2 in range(NW):
        m = jnp.full((16,), w2, jnp.int32) < widv
        pre = pre + jnp.where(m, hist_v[w2], zero)
    start_v[...] = aux_v[2] + pre

    for ev, slv in ((e0_v, sl0_v), (e1_v, sl1_v)):
        for r in range(TN // 16):
            v = ev[pl.ds(r * 16, 16)]
            sg = plsc.load_gather(start_v, [v])
            start = start_v[...]
            rank = zero
            for e in range(E):
                m = v == jnp.int32(e)
                c = plsc.cumsum(jnp.where(m, one, zero))
                rank = jnp.where(m, c - 1, rank)
                pc = plsc.all_reduce_population_count(m)
                start = jnp.where(lane == jnp.int32(e), start + pc, start)
            start_v[...] = start
            slv[pl.ds(r * 16, 16)] = sg + rank

    for r in range(TN // 16):
        tok_v[pl.ds(r * 16, 16)] = jnp.full((16,), base + r * 16, jnp.int32) + lane
    cpx = pltpu.async_copy(x_h.at[tok_v], rows_v, sem)
    cpw = pltpu.async_copy(w0_h.at[tok_v], wv_v, semw)
    cpx.wait()
    cs0 = pltpu.async_copy(rows_v, xd_h.at[sl0_v], sem)
    cs1 = pltpu.async_copy(rows_v, xd_h.at[sl1_v], sem)
    cpw.wait()
    cw0 = pltpu.async_copy(wv_v, wd_h.at[sl0_v], semw)
    cw0.wait()
    cpw2 = pltpu.async_copy(w1_h.at[tok_v], wv2_v, semw)
    cpw2.wait()
    cw1 = pltpu.async_copy(wv2_v, wd_h.at[sl1_v], semw)
    cs0.wait()
    cs1.wait()
    cw1.wait()

    pltpu.sync_copy(sl0_v, s0_h.at[pl.ds(base, TN)])
    pltpu.sync_copy(sl1_v, s1_h.at[pl.ds(base, TN)])


def _ffn_body(be_ref, xd_r, wi_r, wg_r, bi_r, bg_r, wo_r, bo_r, wd_r, out_r):
    hb = pl.program_id(0)
    nb = pl.program_id(1)
    xv = xd_r[...].astype(jnp.bfloat16)
    g = _dotT(xv, wg_r[0].astype(jnp.bfloat16)) + bg_r[0]
    p = _dotT(xv, wi_r[0].astype(jnp.bfloat16)) + bi_r[0]
    h = ((g * _sigmoid(g)) * p).astype(jnp.bfloat16)
    y = _dotT(h, wo_r[0].astype(jnp.bfloat16))
    sl = pl.ds(nb * BN, BN)

    wv = wd_r[:, :1]

    @pl.when(hb == 0)
    def _init():
        out_r[sl, :] = wv * (y + bo_r[0])

    @pl.when(hb > 0)
    def _acc():
        out_r[sl, :] += wv * y


def _grouped_ffn(blk_e, xd, W_in, b_in, W_gate, b_gate, W_out, b_out, wd):
    spec = pltpu.PrefetchScalarGridSpec(
        num_scalar_prefetch=1,
        grid=(HB, NB),
        in_specs=[
            pl.BlockSpec((BN, D), lambda hb, nb, be: (nb, 0)),
            pl.BlockSpec((1, HBS, D), lambda hb, nb, be: (be[nb], hb, 0)),
            pl.BlockSpec((1, HBS, D), lambda hb, nb, be: (be[nb], hb, 0)),
            pl.BlockSpec((1, 1, HBS), lambda hb, nb, be: (be[nb], 0, hb)),
            pl.BlockSpec((1, 1, HBS), lambda hb, nb, be: (be[nb], 0, hb)),
            pl.BlockSpec((1, D, HBS), lambda hb, nb, be: (be[nb], 0, hb)),
            pl.BlockSpec((1, 1, D), lambda hb, nb, be: (be[nb], 0, 0)),
            pl.BlockSpec((BN, 128), lambda hb, nb, be: (nb, 0)),
        ],
        out_specs=pl.BlockSpec((P, D), lambda hb, nb, be: (0, 0)),
    )
    return pl.pallas_call(
        _ffn_body, grid_spec=spec,
        out_shape=jax.ShapeDtypeStruct((P, D), jnp.float32),
    )(blk_e, xd, W_in, W_gate, b_in.reshape(E, 1, H), b_gate.reshape(E, 1, H),
      W_out, b_out.reshape(E, 1, D), wd)


@functools.partial(
    pl.kernel,
    out_type=jax.ShapeDtypeStruct((N, D), jnp.float32),
    mesh=_sc_mesh,
    scratch_types=[
        pltpu.VMEM((TN,), jnp.int32),
        pltpu.VMEM((TN,), jnp.int32),
        pltpu.VMEM((TN, D), jnp.float32),
        pltpu.VMEM((TN, D), jnp.float32),
        pltpu.SemaphoreType.DMA,
        pltpu.SemaphoreType.DMA,
    ],
    compiler_params=pltpu.CompilerParams(needs_layout_passes=False),
)
def _combine(yd_h, s0_h, s1_h, out_h, s0_v, s1_v, r0_v, r1_v, sem0, sem1):
    wid = lax.axis_index("s") * NC + lax.axis_index("c")
    base = wid * TN
    pltpu.sync_copy(s0_h.at[pl.ds(base, TN)], s0_v)
    pltpu.sync_copy(s1_h.at[pl.ds(base, TN)], s1_v)
    cp0 = pltpu.async_copy(yd_h.at[s0_v], r0_v, sem0)
    cp1 = pltpu.async_copy(yd_h.at[s1_v], r1_v, sem1)
    cp0.wait()
    cp1.wait()

    nch = D // 16

    def add_body(i):
        t = i // nch
        c = i % nch
        s = pl.ds(c * 16, 16)
        plsc.addupdate(r0_v.at[t, s], r1_v[t, s])

    plsc.parallel_loop(0, TN * nch, 1, unroll=8)(add_body)
    pltpu.sync_copy(r0_v, out_h.at[pl.ds(base, TN)])


def kernel(x, gate_w, W_in, b_in, W_gate, b_gate, W_out, b_out):
    B, T, C = x.shape
    xf = x.reshape(B * T, C)
    i0, i1, w0, w1, hist, aux, blkeT = _router(xf, gate_w)
    xd, s0, s1, wd = _dispatch(i0.reshape(N), i1.reshape(N), xf, hist, aux,
                               w0, w1)
    yd = _grouped_ffn(blkeT[0], xd, W_in, b_in, W_gate, b_gate, W_out, b_out,
                      wd)
    out = _combine(yd, s0, s1)
    return out.reshape(B, T, C)
